# baseline (speedup 1.0000x reference)
import functools

import jax
import jax.numpy as jnp
from jax import lax
from jax.experimental import pallas as pl
from jax.experimental.pallas import tpu as pltpu
from jax.experimental.pallas import tpu_sc as plsc

ROWS_IN = 500000
ROWS_OUT = 250000
D = 128
C = 400
NCHUNK = ROWS_OUT // C
NC = 2
NS = 16
NW = NC * NS


def _sc_body(x2_hbm, out_hbm, rows0, rows1, gsem, wsem0, wsem1):
    wid = lax.axis_index("s") * NC + lax.axis_index("c")
    niter = (NCHUNK - wid + NW - 1) // NW

    def process(k, rows_v, wsem):
        c = wid + k * NW
        @pl.when(k >= 2)
        def _():
            pltpu.make_async_copy(rows_v, out_hbm.at[pl.ds(0, C)], wsem).wait()

        pltpu.async_copy(
            x2_hbm.at[pl.ds(c * C, C), pl.ds(0, D)], rows_v, gsem
        ).wait()
        pltpu.async_copy(rows_v, out_hbm.at[pl.ds(c * C, C)], wsem)

    def chunk_body(k, _):
        @pl.when(k % 2 == 0)
        def _():
            process(k, rows0, wsem0)

        @pl.when(k % 2 == 1)
        def _():
            process(k, rows1, wsem1)

        return 0

    lax.fori_loop(0, niter, chunk_body, 0)
    pltpu.make_async_copy(rows0, out_hbm.at[pl.ds(0, C)], wsem0).wait()
    pltpu.make_async_copy(rows1, out_hbm.at[pl.ds(0, C)], wsem1).wait()


def kernel(x):
    x2 = x.reshape(ROWS_OUT, 2 * D)
    mesh = plsc.VectorSubcoreMesh(core_axis_name="c", subcore_axis_name="s")
    run = pl.kernel(
        _sc_body,
        mesh=mesh,
        out_type=jax.ShapeDtypeStruct((ROWS_OUT, D), jnp.float32),
        scratch_types=[
            pltpu.VMEM((C, D), jnp.float32),
            pltpu.VMEM((C, D), jnp.float32),
            pltpu.SemaphoreType.DMA,
            pltpu.SemaphoreType.DMA,
            pltpu.SemaphoreType.DMA,
        ],
    )
    return run(x2)


# speedup vs baseline: 3.3999x; 3.3999x over previous
import functools

import jax
import jax.numpy as jnp
from jax import lax
from jax.experimental import pallas as pl
from jax.experimental.pallas import tpu as pltpu
from jax.experimental.pallas import tpu_sc as plsc

ROWS_IN = 500000
ROWS_OUT = 250000
D = 128
L = 16
C = 400
NCHUNK = ROWS_OUT // C
NC = 2
NS = 16
NW = NC * NS


def _sc_body(x_hbm, out_hbm, idx0, idx1, rows0, rows1, gsem0, gsem1, wsem0, wsem1):
    wid = lax.axis_index("s") * NC + lax.axis_index("c")
    niter = (NCHUNK - wid + NW - 1) // NW

    lane2 = 2 * lax.iota(jnp.int32, L)

    def start_gather(c, idx_v, rows_v, gsem):
        base2 = 2 * c * C
        for j in range(C // L):
            idx_v[pl.ds(j * L, L)] = base2 + 2 * j * L + lane2
        pltpu.async_copy(x_hbm.at[idx_v], rows_v, gsem)

    bufs = ((idx0, rows0, gsem0, wsem0), (idx1, rows1, gsem1, wsem1))

    start_gather(wid, idx0, rows0, gsem0)

    def chunk_body(k, _):
        def step(p):
            idx_v, rows_v, gsem, wsem = bufs[p]
            o_idx, o_rows, o_gsem, o_wsem = bufs[1 - p]

            @pl.when(k + 1 < niter)
            def _():
                @pl.when(k >= 1)
                def _():
                    pltpu.make_async_copy(
                        o_rows, out_hbm.at[pl.ds(0, C)], o_wsem
                    ).wait()

                start_gather(wid + (k + 1) * NW, o_idx, o_rows, o_gsem)

            pltpu.make_async_copy(x_hbm.at[idx_v], rows_v, gsem).wait()
            c = wid + k * NW
            pltpu.async_copy(rows_v, out_hbm.at[pl.ds(c * C, C)], wsem)

        @pl.when(k % 2 == 0)
        def _():
            step(0)

        @pl.when(k % 2 == 1)
        def _():
            step(1)

        return 0

    lax.fori_loop(0, niter, chunk_body, 0)
    pltpu.make_async_copy(rows0, out_hbm.at[pl.ds(0, C)], wsem0).wait()
    pltpu.make_async_copy(rows1, out_hbm.at[pl.ds(0, C)], wsem1).wait()


def kernel(x):
    mesh = plsc.VectorSubcoreMesh(core_axis_name="c", subcore_axis_name="s")
    run = pl.kernel(
        _sc_body,
        mesh=mesh,
        out_type=jax.ShapeDtypeStruct((ROWS_OUT, D), jnp.float32),
        scratch_types=[
            pltpu.VMEM((C,), jnp.int32),
            pltpu.VMEM((C,), jnp.int32),
            pltpu.VMEM((C, D), jnp.float32),
            pltpu.VMEM((C, D), jnp.float32),
            pltpu.SemaphoreType.DMA,
            pltpu.SemaphoreType.DMA,
            pltpu.SemaphoreType.DMA,
            pltpu.SemaphoreType.DMA,
        ],
    )
    return run(x)
